# baseline (speedup 1.0000x reference)
import jax
import jax.numpy as jnp
from jax.experimental import pallas as pl
from jax.experimental.pallas import tpu as pltpu

_BM = 200
_NBUF = 5


def _gcn_kernel(w_ref, b_ref, x_hbm, adj_hbm, o_hbm,
                x_vmem, s_ref, out_vmem, bufs, sems, xsem, osem):
    nsteps = adj_hbm.shape[0] // _BM

    def adj_dma(i, slot):
        return pltpu.make_async_copy(
            adj_hbm.at[pl.ds(i * _BM, _BM), :], bufs.at[slot], sems.at[slot])

    def out_dma(i):
        return pltpu.make_async_copy(
            out_vmem.at[pl.ds(i * _BM, _BM), :],
            o_hbm.at[pl.ds(i * _BM, _BM), :], osem)

    for j in range(_NBUF):
        adj_dma(j, j).start()

    xc = pltpu.make_async_copy(x_hbm, x_vmem, xsem)
    xc.start()
    xc.wait()
    s_ref[...] = jnp.dot(x_vmem[...], w_ref[...],
                         preferred_element_type=jnp.float32)

    def outer(oi, carry):
        for j in range(_NBUF):
            i = oi * _NBUF + j
            adj_dma(i, j).wait()
            acc = jnp.dot(bufs[j], s_ref[...],
                          preferred_element_type=jnp.float32)
            out_vmem[pl.ds(i * _BM, _BM), :] = jnp.maximum(
                acc + b_ref[...], 0.0)
            out_dma(i).start()

            @pl.when(i + _NBUF < nsteps)
            def _():
                adj_dma(i + _NBUF, j).start()

        return carry

    jax.lax.fori_loop(0, nsteps // _NBUF, outer, 0)

    def drain(i, carry):
        out_dma(i).wait()
        return carry

    jax.lax.fori_loop(0, nsteps, drain, 0)


def kernel(x, adj, W, b):
    n_rows, d_in = x.shape
    d_out = W.shape[1]
    n_cols = adj.shape[1]

    b2 = b.reshape(1, d_out)
    out = pl.pallas_call(
        _gcn_kernel,
        in_specs=[
            pl.BlockSpec(memory_space=pltpu.VMEM),
            pl.BlockSpec(memory_space=pltpu.VMEM),
            pl.BlockSpec(memory_space=pl.ANY),
            pl.BlockSpec(memory_space=pl.ANY),
        ],
        out_specs=pl.BlockSpec(memory_space=pl.ANY),
        out_shape=jax.ShapeDtypeStruct((n_rows, d_out), jnp.float32),
        scratch_shapes=[
            pltpu.VMEM((n_rows, d_in), jnp.float32),
            pltpu.VMEM((n_cols, d_out), jnp.float32),
            pltpu.VMEM((n_rows, d_out), jnp.float32),
            pltpu.VMEM((_NBUF, _BM, n_cols), jnp.float32),
            pltpu.SemaphoreType.DMA((_NBUF,)),
            pltpu.SemaphoreType.DMA,
            pltpu.SemaphoreType.DMA,
        ],
    )(W, b2, x, adj)
    return out


# speedup vs baseline: 1.0338x; 1.0338x over previous
import jax
import jax.numpy as jnp
from jax.experimental import pallas as pl
from jax.experimental.pallas import tpu as pltpu


def _gcn_kernel(x_ref, w_ref, b_ref, adj_ref, o_ref, s_ref):
    @pl.when(pl.program_id(0) == 0)
    def _():
        s_ref[...] = jnp.dot(x_ref[...], w_ref[...],
                             preferred_element_type=jnp.float32
                             ).astype(jnp.bfloat16)

    acc = jnp.dot(adj_ref[...], s_ref[...],
                  preferred_element_type=jnp.float32)
    o_ref[...] = jnp.maximum(acc + b_ref[...], 0.0)


def kernel(x, adj, W, b):
    n_rows, d_in = x.shape
    d_out = W.shape[1]
    n_cols = adj.shape[1]

    bm = 400
    b2 = b.reshape(1, d_out)
    out = pl.pallas_call(
        _gcn_kernel,
        grid=(pl.cdiv(n_rows, bm),),
        in_specs=[
            pl.BlockSpec((n_rows, d_in), lambda m: (0, 0)),
            pl.BlockSpec((d_in, d_out), lambda m: (0, 0)),
            pl.BlockSpec((1, d_out), lambda m: (0, 0)),
            pl.BlockSpec((bm, n_cols), lambda m: (m, 0)),
        ],
        out_specs=pl.BlockSpec((bm, d_out), lambda m: (m, 0)),
        out_shape=jax.ShapeDtypeStruct((n_rows, d_out), jnp.float32),
        scratch_shapes=[pltpu.VMEM((n_cols, d_out), jnp.bfloat16)],
    )(x, W, b2, adj)
    return out


# speedup vs baseline: 1.0453x; 1.0112x over previous
import jax
import jax.numpy as jnp
from jax.experimental import pallas as pl
from jax.experimental.pallas import tpu as pltpu


def _gcn_kernel(x_ref, w_ref, b_ref, adj_ref, o_ref, s_ref):
    @pl.when(pl.program_id(0) == 0)
    def _():
        s_ref[...] = jnp.dot(x_ref[...], w_ref[...],
                             preferred_element_type=jnp.float32)

    acc = jnp.dot(adj_ref[...], s_ref[...],
                  preferred_element_type=jnp.float32)
    o_ref[...] = jnp.maximum(acc + b_ref[...], 0.0)


def kernel(x, adj, W, b):
    n_rows, d_in = x.shape
    d_out = W.shape[1]
    n_cols = adj.shape[1]

    bm = 400
    b2 = b.reshape(1, d_out)
    out = pl.pallas_call(
        _gcn_kernel,
        grid=(pl.cdiv(n_rows, bm),),
        in_specs=[
            pl.BlockSpec((n_rows, d_in), lambda m: (0, 0)),
            pl.BlockSpec((d_in, d_out), lambda m: (0, 0)),
            pl.BlockSpec((1, d_out), lambda m: (0, 0)),
            pl.BlockSpec((bm, n_cols), lambda m: (m, 0)),
        ],
        out_specs=pl.BlockSpec((bm, d_out), lambda m: (m, 0)),
        out_shape=jax.ShapeDtypeStruct((n_rows, d_out), jnp.float32),
        scratch_shapes=[pltpu.VMEM((n_cols, d_out), jnp.float32)],
    )(x, W, b2, adj)
    return out
